# baseline (speedup 1.0000x reference)
import functools

import jax
import jax.numpy as jnp
from jax import lax
from jax.experimental import pallas as pl
from jax.experimental.pallas import tpu as pltpu
from jax.experimental.pallas import tpu_sc as plsc

N = 10000
D = 256
H = 128
E = 160000
K = 128
NSUB = 16
CH = 80
NCHUNK = NSUB * CH
EPAD = NCHUNK * K - E
NACC = N + 8
NBUF = 2
PH = 2
CPP = CH // PH
ROWS_A = 624
ROWS_TAIL = N - NSUB * ROWS_A
CNT_TILES = 10
CNT_ROWS = N // CNT_TILES

RB = 1000
GRID = N // RB


def _agg_body(compute_counts, *refs):
    if compute_counts:
        (x_lo, x_hi, src2d, dst2d, zrows, zvec,
         out_lo, out_hi, out_cnt,
         src_st, dst_st, ones_v, cz_v, accum, cnt_acc,
         rows, gsems, ssems, csem) = refs
    else:
        (x_lo, x_hi, src2d, dst2d, zrows,
         out_lo, out_hi,
         src_st, dst_st, accum,
         rows, gsems, ssems) = refs

    c = lax.axis_index("c")
    s = lax.axis_index("s")
    r0 = s * ROWS_A
    ch0 = s * CH

    def _rows_copy(src_ref, dst_ref):
        pltpu.sync_copy(src_ref.at[pl.ds(r0, ROWS_A)],
                        dst_ref.at[pl.ds(r0, ROWS_A)])

        @pl.when(s == 0)
        def _():
            pltpu.sync_copy(src_ref.at[pl.ds(NSUB * ROWS_A, ROWS_TAIL)],
                            dst_ref.at[pl.ds(NSUB * ROWS_A, ROWS_TAIL)])

    _rows_copy(zrows, accum)
    if compute_counts:
        @pl.when(jnp.logical_and(c == 0, s < CNT_TILES))
        def _():
            q0 = s * CNT_ROWS
            pltpu.sync_copy(zvec.at[pl.ds(q0, CNT_ROWS)], cz_v)
            pltpu.sync_copy(cz_v, cnt_acc.at[pl.ds(q0, CNT_ROWS)])
        for k in range(K // 16):
            ones_v[pl.ds(k * 16, 16)] = jnp.full((16,), 1.0, jnp.float32)

    def _gather(k, b):
        @pl.when(c == 0)
        def _():
            pltpu.async_copy(x_lo.at[src_st.at[k]], rows[b], gsems[b])

        @pl.when(c == 1)
        def _():
            pltpu.async_copy(x_hi.at[src_st.at[k]], rows[b], gsems[b])

    def _gwait(b):
        pltpu.make_async_copy(x_lo.at[src_st.at[0]], rows[b],
                              gsems[b]).wait()

    def _swait(b):
        pltpu.make_async_copy(rows[b], accum.at[dst_st.at[0]],
                              ssems[b]).wait()

    plsc.subcore_barrier()

    UNROLL = 4

    for p in range(PH):
        pltpu.sync_copy(src2d.at[pl.ds(ch0 + p * CPP, CPP)], src_st)
        pltpu.sync_copy(dst2d.at[pl.ds(ch0 + p * CPP, CPP)], dst_st)

        for q in range(NBUF):
            _gather(q, q)

        def _step(i, carry):
            for ci in range(UNROLL):
                k = i * UNROLL + ci
                b = ci % NBUF

                _gwait(b)
                pltpu.async_copy(rows[b], accum.at[dst_st.at[k]],
                                 ssems[b], add=True)
                if compute_counts:
                    @pl.when(c == 0)
                    def _(k=k):
                        pltpu.async_copy(ones_v, cnt_acc.at[dst_st.at[k]],
                                         csem, add=True)
                _swait(b)

                @pl.when(k + NBUF < CPP)
                def _(k=k, b=b):
                    _gather(k + NBUF, b)
            return carry

        lax.fori_loop(0, CPP // UNROLL, _step, 0)

        if compute_counts:
            @pl.when(c == 0)
            def _():
                pltpu.make_async_copy(dst2d.at[pl.ds(0, CPP)], dst_st,
                                      csem).wait()

    plsc.subcore_barrier()

    @pl.when(c == 0)
    def _():
        _rows_copy(accum, out_lo)

    @pl.when(c == 1)
    def _():
        _rows_copy(accum, out_hi)

    if compute_counts:
        @pl.when(jnp.logical_and(c == 0, s < CNT_TILES))
        def _():
            q0 = s * CNT_ROWS
            pltpu.sync_copy(cnt_acc.at[pl.ds(q0, CNT_ROWS)], cz_v)
            pltpu.sync_copy(cz_v, out_cnt.at[pl.ds(q0, CNT_ROWS)])


def _make_agg(compute_counts):
    out_type = [jax.ShapeDtypeStruct((N, H), jnp.float32),
                jax.ShapeDtypeStruct((N, H), jnp.float32)]
    scratch = [
        pltpu.VMEM((CPP, K), jnp.int32),
        pltpu.VMEM((CPP, K), jnp.int32),
    ]
    if compute_counts:
        out_type.append(jax.ShapeDtypeStruct((N,), jnp.float32))
        scratch.append(pltpu.VMEM((K,), jnp.float32))
        scratch.append(pltpu.VMEM((CNT_ROWS,), jnp.float32))
    scratch.append(pltpu.VMEM_SHARED((NACC, H), jnp.float32))
    if compute_counts:
        scratch.append(pltpu.VMEM_SHARED((NACC,), jnp.float32))
    scratch.append([pltpu.VMEM((K, H), jnp.float32)] * NBUF)
    scratch.append([pltpu.SemaphoreType.DMA] * NBUF)
    scratch.append([pltpu.SemaphoreType.DMA] * NBUF)
    if compute_counts:
        scratch.append(pltpu.SemaphoreType.DMA)
    mesh = plsc.VectorSubcoreMesh(core_axis_name="c", subcore_axis_name="s")
    return pl.kernel(functools.partial(_agg_body, compute_counts),
                     out_type=tuple(out_type), mesh=mesh,
                     scratch_types=scratch)


_agg_cnt = _make_agg(True)
_agg_nocnt = _make_agg(False)


def _dense_body(relu, split, a_lo, a_hi, cnt, r_lo, r_hi, wl, wr, b, *outs):
    inv = 1.0 / jnp.maximum(cnt[...], 1.0)
    wlv = wl[...]
    wrv = wr[...]
    acc = jnp.dot(a_lo[...] * inv, wlv[:H], preferred_element_type=jnp.float32)
    acc += jnp.dot(a_hi[...] * inv, wlv[H:], preferred_element_type=jnp.float32)
    acc += jnp.dot(r_lo[...], wrv[:H], preferred_element_type=jnp.float32)
    acc += jnp.dot(r_hi[...], wrv[H:], preferred_element_type=jnp.float32)
    acc += b[...]
    if relu:
        acc = jnp.maximum(acc, 0.0)
    if split:
        outs[0][...] = acc[:, :H]
        outs[1][...] = acc[:, H:]
    else:
        outs[0][...] = acc


def _make_dense(relu, split):
    in_specs = [
        pl.BlockSpec((RB, H), lambda i: (i, 0)),
        pl.BlockSpec((RB, H), lambda i: (i, 0)),
        pl.BlockSpec((RB, 1), lambda i: (i, 0)),
        pl.BlockSpec((RB, H), lambda i: (i, 0)),
        pl.BlockSpec((RB, H), lambda i: (i, 0)),
        pl.BlockSpec((D, D), lambda i: (0, 0)),
        pl.BlockSpec((D, D), lambda i: (0, 0)),
        pl.BlockSpec((1, D), lambda i: (0, 0)),
    ]
    if split:
        out_specs = [pl.BlockSpec((RB, H), lambda i: (i, 0)),
                     pl.BlockSpec((RB, H), lambda i: (i, 0))]
        out_shape = [jax.ShapeDtypeStruct((N, H), jnp.float32),
                     jax.ShapeDtypeStruct((N, H), jnp.float32)]
    else:
        out_specs = pl.BlockSpec((RB, D), lambda i: (i, 0))
        out_shape = jax.ShapeDtypeStruct((N, D), jnp.float32)
    return pl.pallas_call(functools.partial(_dense_body, relu, split),
                          grid=(GRID,), in_specs=in_specs,
                          out_specs=out_specs, out_shape=out_shape)


_dense_mid = _make_dense(True, True)
_dense_fin = _make_dense(False, False)


def kernel(x, edge_index, W1_l, b1_l, W1_r, W2_l, b2_l, W2_r):
    src2d = jnp.concatenate(
        [edge_index[0].astype(jnp.int32),
         jnp.zeros((EPAD,), jnp.int32)]).reshape(NCHUNK, K)
    dst2d = jnp.concatenate(
        [edge_index[1].astype(jnp.int32),
         jnp.full((EPAD,), N, jnp.int32)]).reshape(NCHUNK, K)
    x_lo = x[:, :H]
    x_hi = x[:, H:]
    zrows = jnp.zeros((N, H), jnp.float32)
    zvec = jnp.zeros((N,), jnp.float32)

    s1_lo, s1_hi, cnt = _agg_cnt(x_lo, x_hi, src2d, dst2d, zrows, zvec)
    cnt2 = cnt.reshape(N, 1)
    h_lo, h_hi = _dense_mid(s1_lo, s1_hi, cnt2, x_lo, x_hi,
                            W1_l.T, W1_r.T, b1_l.reshape(1, D))
    s2_lo, s2_hi = _agg_nocnt(h_lo, h_hi, src2d, dst2d, zrows)
    out = _dense_fin(s2_lo, s2_hi, cnt2, h_lo, h_hi,
                     W2_l.T, W2_r.T, b2_l.reshape(1, D))
    return out


# speedup vs baseline: 1.1780x; 1.1780x over previous
import functools

import jax
import jax.numpy as jnp
from jax import lax
from jax.experimental import pallas as pl
from jax.experimental.pallas import tpu as pltpu
from jax.experimental.pallas import tpu_sc as plsc

N = 10000
D = 256
H = 128
E = 160000
K = 128
NCHUNK = E // K
NSUB = 16
NITER = -(-NCHUNK // NSUB)
ROWS_A = 624
ROWS_TAIL = N - NSUB * ROWS_A
CNT_TILES = 10
CNT_ROWS = N // CNT_TILES

RB = 1000
GRID = N // RB


def _agg_body(compute_counts, *refs):
    if compute_counts:
        (x_lo, x_hi, src, dst,
         out_lo, out_hi, out_cnt,
         src_v, dst_v, rows_v, ones_v, cz_v, accum, cnt_acc, sem) = refs
    else:
        (x_lo, x_hi, src, dst,
         out_lo, out_hi,
         src_v, dst_v, rows_v, accum, sem) = refs

    c = lax.axis_index("c")
    s = lax.axis_index("s")
    r0 = s * ROWS_A

    def _rows_copy(src_ref, dst_ref):
        pltpu.sync_copy(src_ref.at[pl.ds(r0, ROWS_A)],
                        dst_ref.at[pl.ds(r0, ROWS_A)])

        @pl.when(s == 0)
        def _():
            pltpu.sync_copy(src_ref.at[pl.ds(NSUB * ROWS_A, ROWS_TAIL)],
                            dst_ref.at[pl.ds(NSUB * ROWS_A, ROWS_TAIL)])

    z16 = jnp.zeros((16,), jnp.float32)

    def _zrow(r, carry):
        for j in range(H // 16):
            rows_v[r, pl.ds(j * 16, 16)] = z16
        return carry

    lax.fori_loop(0, K, _zrow, 0)
    for t in range(ROWS_A // K):
        pltpu.sync_copy(rows_v, accum.at[pl.ds(r0 + t * K, K)])
    pltpu.sync_copy(rows_v.at[pl.ds(0, ROWS_A % K)],
                    accum.at[pl.ds(r0 + (ROWS_A // K) * K, ROWS_A % K)])

    @pl.when(s == 0)
    def _():
        pltpu.sync_copy(rows_v.at[pl.ds(0, ROWS_TAIL)],
                        accum.at[pl.ds(NSUB * ROWS_A, ROWS_TAIL)])

    if compute_counts:
        @pl.when(jnp.logical_and(c == 0, s < CNT_TILES))
        def _():
            q0 = s * CNT_ROWS
            for j in range(CNT_ROWS // 16):
                cz_v[pl.ds(j * 16, 16)] = z16
            cz_v[pl.ds(CNT_ROWS - 16, 16)] = z16
            pltpu.sync_copy(cz_v, cnt_acc.at[pl.ds(q0, CNT_ROWS)])
        for k in range(K // 16):
            ones_v[pl.ds(k * 16, 16)] = jnp.full((16,), 1.0, jnp.float32)
    plsc.subcore_barrier()

    def step(i, carry):
        j = i * NSUB + s

        @pl.when(j < NCHUNK)
        def _():
            e0 = j * K
            pltpu.sync_copy(src.at[pl.ds(e0, K)], src_v)
            pltpu.sync_copy(dst.at[pl.ds(e0, K)], dst_v)

            @pl.when(c == 0)
            def _():
                pltpu.async_copy(x_lo.at[src_v], rows_v, sem).wait()

            @pl.when(c == 1)
            def _():
                pltpu.async_copy(x_hi.at[src_v], rows_v, sem).wait()

            pltpu.sync_copy(rows_v, accum.at[dst_v], add=True)
            if compute_counts:
                @pl.when(c == 0)
                def _():
                    pltpu.sync_copy(ones_v, cnt_acc.at[dst_v], add=True)

        return carry

    lax.fori_loop(0, NITER, step, 0)
    plsc.subcore_barrier()

    @pl.when(c == 0)
    def _():
        _rows_copy(accum, out_lo)

    @pl.when(c == 1)
    def _():
        _rows_copy(accum, out_hi)

    if compute_counts:
        @pl.when(jnp.logical_and(c == 0, s < CNT_TILES))
        def _():
            q0 = s * CNT_ROWS
            pltpu.sync_copy(cnt_acc.at[pl.ds(q0, CNT_ROWS)], cz_v)
            pltpu.sync_copy(cz_v, out_cnt.at[pl.ds(q0, CNT_ROWS)])


def _make_agg(compute_counts):
    out_type = [jax.ShapeDtypeStruct((N, H), jnp.float32),
                jax.ShapeDtypeStruct((N, H), jnp.float32)]
    scratch = [
        pltpu.VMEM((K,), jnp.int32),
        pltpu.VMEM((K,), jnp.int32),
        pltpu.VMEM((K, H), jnp.float32),
    ]
    if compute_counts:
        out_type.append(jax.ShapeDtypeStruct((N,), jnp.float32))
        scratch.append(pltpu.VMEM((K,), jnp.float32))
        scratch.append(pltpu.VMEM((CNT_ROWS,), jnp.float32))
    scratch.append(pltpu.VMEM_SHARED((N, H), jnp.float32))
    if compute_counts:
        scratch.append(pltpu.VMEM_SHARED((N,), jnp.float32))
    scratch.append(pltpu.SemaphoreType.DMA)
    mesh = plsc.VectorSubcoreMesh(core_axis_name="c", subcore_axis_name="s")
    return pl.kernel(functools.partial(_agg_body, compute_counts),
                     out_type=tuple(out_type), mesh=mesh,
                     scratch_types=scratch)


_agg_cnt = _make_agg(True)
_agg_nocnt = _make_agg(False)


def _dense1_body(a_lo, a_hi, cnt, xin, wl, wr, b, wr2, b2, h_lo, h_hi,
                 root2):
    inv = 1.0 / jnp.maximum(cnt[...], 1.0)
    wlv = wl[...]
    acc = jnp.dot(a_lo[...] * inv, wlv[:H], preferred_element_type=jnp.float32)
    acc += jnp.dot(a_hi[...] * inv, wlv[H:], preferred_element_type=jnp.float32)
    acc += jnp.dot(xin[...], wr[...], preferred_element_type=jnp.float32)
    acc += b[...]
    acc = jnp.maximum(acc, 0.0)
    h_lo[...] = acc[:, :H]
    h_hi[...] = acc[:, H:]
    root2[...] = jnp.dot(acc, wr2[...],
                         preferred_element_type=jnp.float32) + b2[...]


_dense1 = pl.pallas_call(
    _dense1_body, grid=(GRID,),
    in_specs=[pl.BlockSpec((RB, H), lambda i: (i, 0)),
              pl.BlockSpec((RB, H), lambda i: (i, 0)),
              pl.BlockSpec((RB, 1), lambda i: (i, 0)),
              pl.BlockSpec((RB, D), lambda i: (i, 0)),
              pl.BlockSpec((D, D), lambda i: (0, 0)),
              pl.BlockSpec((D, D), lambda i: (0, 0)),
              pl.BlockSpec((1, D), lambda i: (0, 0)),
              pl.BlockSpec((D, D), lambda i: (0, 0)),
              pl.BlockSpec((1, D), lambda i: (0, 0))],
    out_specs=[pl.BlockSpec((RB, H), lambda i: (i, 0)),
               pl.BlockSpec((RB, H), lambda i: (i, 0)),
               pl.BlockSpec((RB, D), lambda i: (i, 0))],
    out_shape=[jax.ShapeDtypeStruct((N, H), jnp.float32),
               jax.ShapeDtypeStruct((N, H), jnp.float32),
               jax.ShapeDtypeStruct((N, D), jnp.float32)])


def _dense2_body(a_lo, a_hi, cnt, root, wl, out):
    inv = 1.0 / jnp.maximum(cnt[...], 1.0)
    wlv = wl[...]
    acc = jnp.dot(a_lo[...] * inv, wlv[:H], preferred_element_type=jnp.float32)
    acc += jnp.dot(a_hi[...] * inv, wlv[H:], preferred_element_type=jnp.float32)
    out[...] = acc + root[...]


_dense2 = pl.pallas_call(
    _dense2_body, grid=(GRID,),
    in_specs=[pl.BlockSpec((RB, H), lambda i: (i, 0)),
              pl.BlockSpec((RB, H), lambda i: (i, 0)),
              pl.BlockSpec((RB, 1), lambda i: (i, 0)),
              pl.BlockSpec((RB, D), lambda i: (i, 0)),
              pl.BlockSpec((D, D), lambda i: (0, 0))],
    out_specs=pl.BlockSpec((RB, D), lambda i: (i, 0)),
    out_shape=jax.ShapeDtypeStruct((N, D), jnp.float32))


def kernel(x, edge_index, W1_l, b1_l, W1_r, W2_l, b2_l, W2_r):
    src = edge_index[0].astype(jnp.int32)
    dst = edge_index[1].astype(jnp.int32)
    x_lo = x[:, :H]
    x_hi = x[:, H:]

    s1_lo, s1_hi, cnt = _agg_cnt(x_lo, x_hi, src, dst)
    cnt2 = cnt.reshape(N, 1)
    h_lo, h_hi, root2 = _dense1(s1_lo, s1_hi, cnt2, x, W1_l.T, W1_r.T,
                                b1_l.reshape(1, D), W2_r.T,
                                b2_l.reshape(1, D))
    s2_lo, s2_hi = _agg_nocnt(h_lo, h_hi, src, dst)
    out = _dense2(s2_lo, s2_hi, cnt2, root2, W2_l.T)
    return out


# speedup vs baseline: 1.5758x; 1.3377x over previous
import functools

import jax
import jax.numpy as jnp
from jax import lax
from jax.experimental import pallas as pl
from jax.experimental.pallas import tpu as pltpu
from jax.experimental.pallas import tpu_sc as plsc

N = 10000
D = 256
H = 128
E = 160000
K = 128
NCHUNK = E // K
NSUB = 16
NITER = -(-NCHUNK // NSUB)
ROWS_A = 624
ROWS_TAIL = N - NSUB * ROWS_A
CNT_TILES = 10
CNT_ROWS = N // CNT_TILES

RB = 1000
GRID = N // RB


def _agg_body(compute_counts, *refs):
    if compute_counts:
        (x_lo, x_hi, src, dst,
         out_lo, out_hi, out_cnt,
         src_v, dst_v, rows_v, ones_v, cz_v, accum, cnt_acc, sem,
         isems) = refs
    else:
        (x_lo, x_hi, src, dst,
         out_lo, out_hi,
         src_v, dst_v, rows_v, accum, sem, isems) = refs

    c = lax.axis_index("c")
    s = lax.axis_index("s")
    r0 = s * ROWS_A

    def _rows_copy(src_ref, dst_ref):
        pltpu.sync_copy(src_ref.at[pl.ds(r0, ROWS_A)],
                        dst_ref.at[pl.ds(r0, ROWS_A)])

        @pl.when(s == 0)
        def _():
            pltpu.sync_copy(src_ref.at[pl.ds(NSUB * ROWS_A, ROWS_TAIL)],
                            dst_ref.at[pl.ds(NSUB * ROWS_A, ROWS_TAIL)])

    z16 = jnp.zeros((16,), jnp.float32)

    def _zrow(r, carry):
        for j in range(H // 16):
            rows_v[r, pl.ds(j * 16, 16)] = z16
        return carry

    lax.fori_loop(0, K, _zrow, 0)
    for t in range(ROWS_A // K):
        pltpu.sync_copy(rows_v, accum.at[pl.ds(r0 + t * K, K)])
    pltpu.sync_copy(rows_v.at[pl.ds(0, ROWS_A % K)],
                    accum.at[pl.ds(r0 + (ROWS_A // K) * K, ROWS_A % K)])

    @pl.when(s == 0)
    def _():
        pltpu.sync_copy(rows_v.at[pl.ds(0, ROWS_TAIL)],
                        accum.at[pl.ds(NSUB * ROWS_A, ROWS_TAIL)])

    if compute_counts:
        @pl.when(jnp.logical_and(c == 0, s < CNT_TILES))
        def _():
            q0 = s * CNT_ROWS
            for j in range(CNT_ROWS // 16):
                cz_v[pl.ds(j * 16, 16)] = z16
            cz_v[pl.ds(CNT_ROWS - 16, 16)] = z16
            pltpu.sync_copy(cz_v, cnt_acc.at[pl.ds(q0, CNT_ROWS)])
        for k in range(K // 16):
            ones_v[pl.ds(k * 16, 16)] = jnp.full((16,), 1.0, jnp.float32)
    plsc.subcore_barrier()

    def _iload(j, b):
        pltpu.async_copy(src.at[pl.ds(j * K, K)], src_v[b], isems[2 * b])
        pltpu.async_copy(dst.at[pl.ds(j * K, K)], dst_v[b], isems[2 * b + 1])

    def _iwait(b):
        pltpu.make_async_copy(src.at[pl.ds(0, K)], src_v[b],
                              isems[2 * b]).wait()
        pltpu.make_async_copy(dst.at[pl.ds(0, K)], dst_v[b],
                              isems[2 * b + 1]).wait()

    _iload(s, 0)

    def step(i, carry):
        for half in range(2):
            j = (2 * i + half) * NSUB + s
            b = half
            nb = 1 - half

            @pl.when(j < NCHUNK)
            def _(j=j, b=b, nb=nb):
                jn = j + NSUB

                @pl.when(jn < NCHUNK)
                def _(jn=jn, nb=nb):
                    _iload(jn, nb)

                _iwait(b)

                @pl.when(c == 0)
                def _(b=b):
                    pltpu.async_copy(x_lo.at[src_v[b]], rows_v, sem).wait()

                @pl.when(c == 1)
                def _(b=b):
                    pltpu.async_copy(x_hi.at[src_v[b]], rows_v, sem).wait()

                pltpu.sync_copy(rows_v, accum.at[dst_v[b]], add=True)
                if compute_counts:
                    @pl.when(c == 0)
                    def _(b=b):
                        pltpu.sync_copy(ones_v, cnt_acc.at[dst_v[b]],
                                        add=True)

        return carry

    lax.fori_loop(0, (NITER + 1) // 2, step, 0)
    plsc.subcore_barrier()

    @pl.when(c == 0)
    def _():
        _rows_copy(accum, out_lo)

    @pl.when(c == 1)
    def _():
        _rows_copy(accum, out_hi)

    if compute_counts:
        @pl.when(jnp.logical_and(c == 0, s < CNT_TILES))
        def _():
            q0 = s * CNT_ROWS
            pltpu.sync_copy(cnt_acc.at[pl.ds(q0, CNT_ROWS)], cz_v)
            pltpu.sync_copy(cz_v, out_cnt.at[pl.ds(q0, CNT_ROWS)])


def _make_agg(compute_counts):
    out_type = [jax.ShapeDtypeStruct((N, H), jnp.float32),
                jax.ShapeDtypeStruct((N, H), jnp.float32)]
    scratch = [
        [pltpu.VMEM((K,), jnp.int32)] * 2,
        [pltpu.VMEM((K,), jnp.int32)] * 2,
        pltpu.VMEM((K, H), jnp.float32),
    ]
    if compute_counts:
        out_type.append(jax.ShapeDtypeStruct((N,), jnp.float32))
        scratch.append(pltpu.VMEM((K,), jnp.float32))
        scratch.append(pltpu.VMEM((CNT_ROWS,), jnp.float32))
    scratch.append(pltpu.VMEM_SHARED((N, H), jnp.float32))
    if compute_counts:
        scratch.append(pltpu.VMEM_SHARED((N,), jnp.float32))
    scratch.append(pltpu.SemaphoreType.DMA)
    scratch.append([pltpu.SemaphoreType.DMA] * 4)
    mesh = plsc.VectorSubcoreMesh(core_axis_name="c", subcore_axis_name="s")
    return pl.kernel(functools.partial(_agg_body, compute_counts),
                     out_type=tuple(out_type), mesh=mesh,
                     scratch_types=scratch)


_agg_cnt = _make_agg(True)
_agg_nocnt = _make_agg(False)


def _dense1_body(a_lo, a_hi, cnt, xin, wl, wr, b, wr2, b2, h_lo, h_hi,
                 root2):
    inv = 1.0 / jnp.maximum(cnt[...], 1.0)
    wlv = wl[...]
    acc = jnp.dot(a_lo[...] * inv, wlv[:H], preferred_element_type=jnp.float32)
    acc += jnp.dot(a_hi[...] * inv, wlv[H:], preferred_element_type=jnp.float32)
    acc += jnp.dot(xin[...], wr[...], preferred_element_type=jnp.float32)
    acc += b[...]
    acc = jnp.maximum(acc, 0.0)
    h_lo[...] = acc[:, :H]
    h_hi[...] = acc[:, H:]
    root2[...] = jnp.dot(acc, wr2[...],
                         preferred_element_type=jnp.float32) + b2[...]


_dense1 = pl.pallas_call(
    _dense1_body, grid=(GRID,),
    in_specs=[pl.BlockSpec((RB, H), lambda i: (i, 0)),
              pl.BlockSpec((RB, H), lambda i: (i, 0)),
              pl.BlockSpec((RB, 1), lambda i: (i, 0)),
              pl.BlockSpec((RB, D), lambda i: (i, 0)),
              pl.BlockSpec((D, D), lambda i: (0, 0)),
              pl.BlockSpec((D, D), lambda i: (0, 0)),
              pl.BlockSpec((1, D), lambda i: (0, 0)),
              pl.BlockSpec((D, D), lambda i: (0, 0)),
              pl.BlockSpec((1, D), lambda i: (0, 0))],
    out_specs=[pl.BlockSpec((RB, H), lambda i: (i, 0)),
               pl.BlockSpec((RB, H), lambda i: (i, 0)),
               pl.BlockSpec((RB, D), lambda i: (i, 0))],
    out_shape=[jax.ShapeDtypeStruct((N, H), jnp.float32),
               jax.ShapeDtypeStruct((N, H), jnp.float32),
               jax.ShapeDtypeStruct((N, D), jnp.float32)])


def _dense2_body(a_lo, a_hi, cnt, root, wl, out):
    inv = 1.0 / jnp.maximum(cnt[...], 1.0)
    wlv = wl[...]
    acc = jnp.dot(a_lo[...] * inv, wlv[:H], preferred_element_type=jnp.float32)
    acc += jnp.dot(a_hi[...] * inv, wlv[H:], preferred_element_type=jnp.float32)
    out[...] = acc + root[...]


_dense2 = pl.pallas_call(
    _dense2_body, grid=(GRID,),
    in_specs=[pl.BlockSpec((RB, H), lambda i: (i, 0)),
              pl.BlockSpec((RB, H), lambda i: (i, 0)),
              pl.BlockSpec((RB, 1), lambda i: (i, 0)),
              pl.BlockSpec((RB, D), lambda i: (i, 0)),
              pl.BlockSpec((D, D), lambda i: (0, 0))],
    out_specs=pl.BlockSpec((RB, D), lambda i: (i, 0)),
    out_shape=jax.ShapeDtypeStruct((N, D), jnp.float32))


def kernel(x, edge_index, W1_l, b1_l, W1_r, W2_l, b2_l, W2_r):
    src = edge_index[0].astype(jnp.int32)
    dst = edge_index[1].astype(jnp.int32)
    x_lo = x[:, :H]
    x_hi = x[:, H:]

    s1_lo, s1_hi, cnt = _agg_cnt(x_lo, x_hi, src, dst)
    cnt2 = cnt.reshape(N, 1)
    h_lo, h_hi, root2 = _dense1(s1_lo, s1_hi, cnt2, x, W1_l.T, W1_r.T,
                                b1_l.reshape(1, D), W2_r.T,
                                b2_l.reshape(1, D))
    s2_lo, s2_hi = _agg_nocnt(h_lo, h_hi, src, dst)
    out = _dense2(s2_lo, s2_hi, cnt2, root2, W2_l.T)
    return out
